# baseline (speedup 1.0000x reference)
import functools

import jax
import jax.numpy as jnp
from jax import lax
from jax.experimental import pallas as pl
from jax.experimental.pallas import tpu as pltpu
from jax.experimental.pallas import tpu_sc as plsc

_V = 65536
_NNZ = 4194304
_ALPHA = 0.5
_NC = 2
_NS = 16
_NW = _NC * _NS
_NNZ_W = _NNZ // _NW
_CHUNK = 4096
_NCHUNK = _NNZ_W // _CHUNK
_GROUPS = _CHUNK // 16
_UNROLL = 8
_LCHUNK = _CHUNK
_NLCHUNK = _V // _LCHUNK

_mesh = plsc.VectorSubcoreMesh(core_axis_name="c", subcore_axis_name="s")


@functools.partial(
    pl.kernel,
    out_type=jax.ShapeDtypeStruct((_NW, _V), jnp.float32),
    mesh=_mesh,
    scratch_types=[
        pltpu.VMEM((_V // 2,), jnp.int32),
        pltpu.VMEM((_V,), jnp.float32),
        pltpu.VMEM((2, _CHUNK), jnp.int32),
        pltpu.VMEM((2, _CHUNK), jnp.int32),
        pltpu.VMEM((2, _CHUNK), jnp.float32),
        pltpu.SemaphoreType.DMA,
        pltpu.SemaphoreType.DMA,
        pltpu.SemaphoreType.DMA,
        pltpu.SemaphoreType.DMA,
    ],
    compiler_params=pltpu.CompilerParams(needs_layout_passes=False, use_tc_tiling_on_sc=False, disable_bounds_checks=True, disable_semaphore_checks=True),
)
def _spmv_sc(logits_hbm, rows_hbm, cols_hbm, vals_hbm, out_hbm,
             plog, acc, rbuf, cbuf, vbuf, sem0, sem1, sem2, sem3):
    wid = lax.axis_index("s") * _NC + lax.axis_index("c")
    base = wid * _NNZ_W
    sems = (sem0, sem1)
    lsems = (sem2, sem3)

    def _copies(ci, slot):
        off = base + ci * _CHUNK
        sem = sems[slot]
        return (
            pltpu.make_async_copy(rows_hbm.at[pl.ds(off, _CHUNK)],
                                  rbuf.at[slot], sem),
            pltpu.make_async_copy(cols_hbm.at[pl.ds(off, _CHUNK)],
                                  cbuf.at[slot], sem),
            pltpu.make_async_copy(vals_hbm.at[pl.ds(off, _CHUNK)],
                                  vbuf.at[slot], sem),
        )

    def _start(ci, slot):
        for d in _copies(ci, slot):
            d.start()

    def _wait(ci, slot):
        for d in _copies(ci, slot):
            d.wait()

    def _lcopy(k, slot):
        koff = lax.rem(k + wid, _NLCHUNK)
        return koff, pltpu.make_async_copy(
            logits_hbm.at[pl.ds(koff * _LCHUNK, _LCHUNK)],
            vbuf.at[slot], lsems[slot])

    _lcopy(0, 0)[1].start()
    half = jnp.int32(0x8000)
    himask = jnp.int32(-65536)
    for k in range(_NLCHUNK):
        slot = k & 1
        if k + 1 < _NLCHUNK:
            _lcopy(k + 1, 1 - slot)[1].start()
        koff, d = _lcopy(k, slot)
        d.wait()

        pbase = koff * (_LCHUNK // 2)

        @plsc.parallel_loop(0, _LCHUNK // 32, 1, unroll=8)
        def _pk(m):
            a = plsc.bitcast(vbuf[slot, pl.ds(m * 32, 16)], jnp.int32)
            b = plsc.bitcast(vbuf[slot, pl.ds(m * 32 + 16, 16)], jnp.int32)
            w = jnp.bitwise_or(
                jnp.bitwise_and(b + half, himask),
                lax.shift_right_logical(a + half, 16))
            plog[pl.ds(pbase + m * 16, 16)] = w

    _start(0, 0)
    _start(1, 1)

    zero = jnp.zeros((16,), jnp.float32)

    @plsc.parallel_loop(0, _V // 16, 1, unroll=8)
    def _zero(i):
        acc[pl.ds(i * 16, 16)] = zero

    def _compute(slot):
        @plsc.parallel_loop(0, _GROUPS, 1, unroll=_UNROLL)
        def _grp(g):
            s = g * 16
            c16 = cbuf[slot, pl.ds(s, 16)]
            r16 = rbuf[slot, pl.ds(s, 16)]
            v16 = vbuf[slot, pl.ds(s, 16)]
            idx = jnp.bitwise_or(
                jnp.bitwise_and(lax.shift_right_logical(c16, 1),
                                jnp.int32(-16)),
                jnp.bitwise_and(c16, jnp.int32(15)))
            w = plsc.load_gather(plog, [idx])
            hi = jnp.bitwise_and(w, himask)
            lo = lax.shift_left(w, 16)
            bits = jnp.where(jnp.bitwise_and(c16, jnp.int32(16)) != 0, hi, lo)
            lg = plsc.bitcast(bits, jnp.float32)
            plsc.addupdate_scatter(acc, [r16], lg * v16)

    def _pair(pi, c):
        ci0 = pi * 2
        _wait(ci0, 0)
        _compute(0)

        @pl.when(ci0 + 2 < _NCHUNK)
        def _():
            _start(ci0 + 2, 0)

        _wait(ci0 + 1, 1)
        _compute(1)

        @pl.when(ci0 + 3 < _NCHUNK)
        def _():
            _start(ci0 + 3, 1)

        return c

    lax.fori_loop(0, _NCHUNK // 2, _pair, 0)
    pltpu.sync_copy(acc, out_hbm.at[wid])


def _combine_body(p_ref, l_ref, o_ref):
    o_ref[...] = jnp.sum(p_ref[...], axis=0) - _ALPHA * l_ref[...]


_combine = pl.pallas_call(
    _combine_body,
    out_shape=jax.ShapeDtypeStruct((_V,), jnp.float32),
)


def kernel(logits, S_rows, S_cols, S_vals):
    partials = _spmv_sc(logits, S_rows, S_cols, S_vals)
    return _combine(partials, logits)


# speedup vs baseline: 1.0037x; 1.0037x over previous
import functools

import jax
import jax.numpy as jnp
from jax import lax
from jax.experimental import pallas as pl
from jax.experimental.pallas import tpu as pltpu
from jax.experimental.pallas import tpu_sc as plsc

_V = 65536
_NNZ = 4194304
_ALPHA = 0.5
_NC = 2
_NS = 16
_NW = _NC * _NS
_NNZ_W = _NNZ // _NW
_CHUNK = 4096
_NCHUNK = _NNZ_W // _CHUNK
_GROUPS = _CHUNK // 16
_UNROLL = 8
_LCHUNK = _CHUNK
_NLCHUNK = _V // _LCHUNK

_mesh = plsc.VectorSubcoreMesh(core_axis_name="c", subcore_axis_name="s")


@functools.partial(
    pl.kernel,
    out_type=jax.ShapeDtypeStruct((_NW, _V), jnp.float32),
    mesh=_mesh,
    scratch_types=[
        pltpu.VMEM((_V // 2,), jnp.int32),
        pltpu.VMEM((_V,), jnp.float32),
        pltpu.VMEM((2, _CHUNK), jnp.int32),
        pltpu.VMEM((2, _CHUNK), jnp.int32),
        pltpu.VMEM((2, _CHUNK), jnp.float32),
        pltpu.SemaphoreType.DMA,
        pltpu.SemaphoreType.DMA,
        pltpu.SemaphoreType.DMA,
        pltpu.SemaphoreType.DMA,
    ],
    compiler_params=pltpu.CompilerParams(needs_layout_passes=False, use_tc_tiling_on_sc=False),
)
def _spmv_sc(logits_hbm, rows_hbm, cols_hbm, vals_hbm, out_hbm,
             plog, acc, rbuf, cbuf, vbuf, sem0, sem1, sem2, sem3):
    wid = lax.axis_index("s") * _NC + lax.axis_index("c")
    base = wid * _NNZ_W
    sems = (sem0, sem1)
    lsems = (sem2, sem3)

    def _copies(ci, slot):
        off = base + ci * _CHUNK
        sem = sems[slot]
        return (
            pltpu.make_async_copy(rows_hbm.at[pl.ds(off, _CHUNK)],
                                  rbuf.at[slot], sem),
            pltpu.make_async_copy(cols_hbm.at[pl.ds(off, _CHUNK)],
                                  cbuf.at[slot], sem),
            pltpu.make_async_copy(vals_hbm.at[pl.ds(off, _CHUNK)],
                                  vbuf.at[slot], sem),
        )

    def _start(ci, slot):
        for d in _copies(ci, slot):
            d.start()

    def _wait(ci, slot):
        for d in _copies(ci, slot):
            d.wait()

    def _lcopy(k, slot):
        koff = lax.rem(k + wid, _NLCHUNK)
        return koff, pltpu.make_async_copy(
            logits_hbm.at[pl.ds(koff * _LCHUNK, _LCHUNK)],
            vbuf.at[slot], lsems[slot])

    _lcopy(0, 0)[1].start()
    half = jnp.int32(0x8000)
    himask = jnp.int32(-65536)
    for k in range(_NLCHUNK):
        slot = k & 1
        if k + 1 < _NLCHUNK:
            _lcopy(k + 1, 1 - slot)[1].start()
        koff, d = _lcopy(k, slot)
        d.wait()

        pbase = koff * (_LCHUNK // 2)

        @plsc.parallel_loop(0, _LCHUNK // 32, 1, unroll=8)
        def _pk(m):
            a = plsc.bitcast(vbuf[slot, pl.ds(m * 32, 16)], jnp.int32)
            b = plsc.bitcast(vbuf[slot, pl.ds(m * 32 + 16, 16)], jnp.int32)
            w = jnp.bitwise_or(
                jnp.bitwise_and(b + half, himask),
                lax.shift_right_logical(a + half, 16))
            plog[pl.ds(pbase + m * 16, 16)] = w

    _start(0, 0)
    _start(1, 1)

    zero = jnp.zeros((16,), jnp.float32)

    @plsc.parallel_loop(0, _V // 16, 1, unroll=8)
    def _zero(i):
        acc[pl.ds(i * 16, 16)] = zero

    def _compute(slot):
        @plsc.parallel_loop(0, _GROUPS, 1, unroll=_UNROLL)
        def _grp(g):
            s = g * 16
            c16 = cbuf[slot, pl.ds(s, 16)]
            r16 = rbuf[slot, pl.ds(s, 16)]
            v16 = vbuf[slot, pl.ds(s, 16)]
            idx = jnp.bitwise_or(
                jnp.bitwise_and(lax.shift_right_logical(c16, 1),
                                jnp.int32(-16)),
                jnp.bitwise_and(c16, jnp.int32(15)))
            w = plsc.load_gather(plog, [idx])
            hi = jnp.bitwise_and(w, himask)
            lo = lax.shift_left(w, 16)
            bits = jnp.where(jnp.bitwise_and(c16, jnp.int32(16)) != 0, hi, lo)
            lg = plsc.bitcast(bits, jnp.float32)
            plsc.addupdate_scatter(acc, [r16], lg * v16)

    def _pair(pi, c):
        ci0 = pi * 2
        _wait(ci0, 0)
        _compute(0)

        @pl.when(ci0 + 2 < _NCHUNK)
        def _():
            _start(ci0 + 2, 0)

        _wait(ci0 + 1, 1)
        _compute(1)

        @pl.when(ci0 + 3 < _NCHUNK)
        def _():
            _start(ci0 + 3, 1)

        return c

    lax.fori_loop(0, _NCHUNK // 2, _pair, 0)
    pltpu.sync_copy(acc, out_hbm.at[wid])


def _combine_body(p_ref, l_ref, o_ref):
    o_ref[...] = jnp.sum(p_ref[...], axis=0) - _ALPHA * l_ref[...]


_combine = pl.pallas_call(
    _combine_body,
    out_shape=jax.ShapeDtypeStruct((_V,), jnp.float32),
)


def kernel(logits, S_rows, S_cols, S_vals):
    partials = _spmv_sc(logits, S_rows, S_cols, S_vals)
    return _combine(partials, logits)
